# Initial kernel scaffold; baseline (speedup 1.0000x reference)
import functools

import jax
import jax.numpy as jnp
from jax import lax
from jax.experimental import pallas as pl
from jax.experimental.pallas import tpu as pltpu
from jax.experimental.pallas import tpu_sc as plsc

EPS = 1e-15
BOHR2 = 0.529177 * 0.529177
KE_KCAL = 332.0637
R_ON = 1.25
R_OFF = 3.75
INV_W = 1.0 / (R_OFF - R_ON)

FEAT = 128
NMOL = 448

NC, NS, L = 2, 16, 16
NW = NC * NS

R = 1024
CHUNK = 1024
SUB = 128
GSUB = CHUNK // SUB
GROUPS = CHUNK // L


def _rsqrt(s):
    i = lax.bitcast_convert_type(s, jnp.int32)
    i = 0x5F3759DF - lax.shift_right_arithmetic(i, 1)
    y = lax.bitcast_convert_type(i, jnp.float32)
    for _ in range(3):
        y = y * (1.5 - 0.5 * s * y * y)
    return y


def _charge_body(nblk, f_ref, z_ref, mol_ref, tc_ref, na_ref, w_ref,
                 zt_ref, charge_ref, molsum_ref, corr_ref):
    pid = pl.program_id(0)
    f = f_ref[...]
    zcol = z_ref[...].reshape(R, 1)
    onehot_z = (zcol == lax.broadcasted_iota(jnp.int32, (R, FEAT), 1)
                ).astype(jnp.float32)
    charge = (jnp.sum(f * w_ref[...], axis=1, keepdims=True)
              + jax.lax.dot(onehot_z, zt_ref[...],
                            precision=jax.lax.Precision.HIGHEST))
    charge_ref[...] = charge

    molcol = mol_ref[...].reshape(R, 1)
    onehot_m = (molcol == lax.broadcasted_iota(jnp.int32, (R, NMOL), 1)
                ).astype(jnp.float32)
    part = jax.lax.dot(charge.reshape(1, R), onehot_m,
                       precision=jax.lax.Precision.HIGHEST)

    @pl.when(pid == 0)
    def _():
        molsum_ref[...] = jnp.zeros_like(molsum_ref)

    molsum_ref[...] += part

    @pl.when(pid == nblk - 1)
    def _():
        denom = jnp.maximum(na_ref[...], 1).astype(jnp.float32)
        corr_ref[...] = (tc_ref[...] - molsum_ref[...]) / denom


def _q_body(n_rows, charge_ref, mol_ref, xyz_ref, corr_ref, q_ref, rec_ref):
    pid = pl.program_id(0)
    molcol = mol_ref[...].reshape(R, 1)
    onehot_m = (molcol == lax.broadcasted_iota(jnp.int32, (R, NMOL), 1)
                ).astype(jnp.float32)
    q = charge_ref[...] + jax.lax.dot(
        onehot_m, corr_ref[...].reshape(NMOL, 1),
        precision=jax.lax.Precision.HIGHEST)
    rowid = pid * R + lax.broadcasted_iota(jnp.int32, (R, 1), 0)
    qm = jnp.where(rowid < n_rows, q, 0.0)
    q_ref[...] = qm
    rec_ref[...] = jnp.concatenate(
        [xyz_ref[...], qm, molcol.astype(jnp.float32),
         jnp.zeros((R, 11), jnp.float32)], axis=1)


def _edge_body(nchunk, ii_hbm, jj_hbm, recs_hbm, out_hbm,
               idx_i, idx_j, ri, rj, acc, partial, sem_a, sem_b):
    wid = lax.axis_index("s") * NC + lax.axis_index("c")
    base_row = wid * (nchunk * GSUB)
    lane = lax.iota(jnp.int32, L)

    zeros16 = jnp.zeros((L,), jnp.float32)

    def zero_body(m, carry):
        acc[m] = zeros16
        return carry

    lax.fori_loop(0, NMOL, zero_body, 0)

    def chunk_body(c, carry):
        r0 = base_row + c * GSUB
        pltpu.sync_copy(ii_hbm.at[pl.ds(r0, GSUB)], idx_i)
        pltpu.sync_copy(jj_hbm.at[pl.ds(r0, GSUB)], idx_j)
        cps = []
        for k in range(GSUB):
            cps.append(pltpu.async_copy(
                recs_hbm.at[idx_i.at[k]], ri.at[pl.ds(k * SUB, SUB)], sem_a))
            cps.append(pltpu.async_copy(
                recs_hbm.at[idx_j.at[k]], rj.at[pl.ds(k * SUB, SUB)], sem_b))
        for cp in cps:
            cp.wait()

        def group_body(g, gcarry):
            row = g * L + lane

            def col(ref, cix):
                return plsc.load_gather(ref, [row, lane * 0 + cix])

            xi = col(ri, 0); yi = col(ri, 1); zi = col(ri, 2)
            qi = col(ri, 3); mi = col(ri, 4)
            xj = col(rj, 0); yj = col(rj, 1); zj = col(rj, 2)
            qj = col(rj, 3)

            dx = xi - xj
            dy = yi - yj
            dz = zi - zj
            s = dx * dx + dy * dy + dz * dz + EPS
            t = _rsqrt(s)
            r = s * t
            u = _rsqrt(s + BOHR2)

            x = (r - R_ON) * INV_W
            y = 1.0 - x
            mask = (x > 0.0) & (y > 0.0)
            denom = jnp.where(mask, x * y, 1.0)
            earg = (x - y) / denom
            small = mask & (earg < 34.0)
            safe = jnp.where(small, earg, 0.0)
            mid = jnp.where(earg >= 34.0, 0.0, 1.0 / (1.0 + jnp.exp(safe)))
            fs = jnp.where(mask, mid,
                           jnp.where((x <= 0.0) & (y > 0.0), 1.0, 0.0))

            pw = KE_KCAL * (qi * qj) * (fs * u + (1.0 - fs) * t)
            seg = mi.astype(jnp.int32)
            plsc.addupdate_scatter(acc, [seg, lane], pw)
            return gcarry

        lax.fori_loop(0, GROUPS, group_body, 0)
        return carry

    lax.fori_loop(0, nchunk, chunk_body, 0)

    def red_body(m, carry):
        partial[m] = jnp.sum(acc[m])
        return carry

    lax.fori_loop(0, NMOL, red_body, 0)
    pltpu.sync_copy(partial, out_hbm.at[wid])


def _combine_body(p_ref, out_ref):
    out_ref[...] = jnp.sum(p_ref[...], axis=0, keepdims=True)


def kernel(f, z, xyz, total_charge, num_atoms, mol_nbrs, W, z_table):
    n = f.shape[0]
    e = mol_nbrs.shape[0]
    npad = ((n + 1 + R - 1) // R) * R
    nblk = npad // R
    per_tile = (e + NW * CHUNK - 1) // (NW * CHUNK) * CHUNK
    epad = per_tile * NW
    nchunk = per_tile // CHUNK

    mol_idx = jnp.repeat(jnp.arange(NMOL, dtype=jnp.int32), num_atoms,
                         total_repeat_length=n)
    f_p = jnp.pad(f, ((0, npad - n), (0, 0)))
    z_p = jnp.pad(z, (0, npad - n)).reshape(nblk, 1, R)
    mol_p = jnp.pad(mol_idx, (0, npad - n)).reshape(nblk, 1, R)
    xyz_p = jnp.pad(xyz, ((0, npad - n), (0, 0)))
    zt_p = jnp.zeros((FEAT, 1), jnp.float32).at[:z_table.shape[0]].set(z_table)
    tc2 = total_charge.reshape(1, NMOL)
    na2 = num_atoms.reshape(1, NMOL)

    grid_a = pl.pallas_call(
        functools.partial(_charge_body, nblk),
        grid=(nblk,),
        in_specs=[
            pl.BlockSpec((R, FEAT), lambda i: (i, 0)),
            pl.BlockSpec((1, 1, R), lambda i: (i, 0, 0)),
            pl.BlockSpec((1, 1, R), lambda i: (i, 0, 0)),
            pl.BlockSpec((1, NMOL), lambda i: (0, 0)),
            pl.BlockSpec((1, NMOL), lambda i: (0, 0)),
            pl.BlockSpec((1, FEAT), lambda i: (0, 0)),
            pl.BlockSpec((FEAT, 1), lambda i: (0, 0)),
        ],
        out_specs=[
            pl.BlockSpec((R, 1), lambda i: (i, 0)),
            pl.BlockSpec((1, NMOL), lambda i: (0, 0)),
            pl.BlockSpec((1, NMOL), lambda i: (0, 0)),
        ],
        out_shape=[
            jax.ShapeDtypeStruct((npad, 1), jnp.float32),
            jax.ShapeDtypeStruct((1, NMOL), jnp.float32),
            jax.ShapeDtypeStruct((1, NMOL), jnp.float32),
        ],
    )
    charge, _, corr = grid_a(f_p, z_p, mol_p, tc2, na2, W, zt_p)

    grid_b = pl.pallas_call(
        functools.partial(_q_body, n),
        grid=(nblk,),
        in_specs=[
            pl.BlockSpec((R, 1), lambda i: (i, 0)),
            pl.BlockSpec((1, 1, R), lambda i: (i, 0, 0)),
            pl.BlockSpec((R, 3), lambda i: (i, 0)),
            pl.BlockSpec((1, NMOL), lambda i: (0, 0)),
        ],
        out_specs=[
            pl.BlockSpec((R, 1), lambda i: (i, 0)),
            pl.BlockSpec((R, 16), lambda i: (i, 0)),
        ],
        out_shape=[
            jax.ShapeDtypeStruct((npad, 1), jnp.float32),
            jax.ShapeDtypeStruct((npad, 16), jnp.float32),
        ],
    )
    q_pad, recs = grid_b(charge, mol_p, xyz_p, corr)

    ii = jnp.concatenate(
        [mol_nbrs[:, 0], jnp.full((epad - e,), n, jnp.int32)]
    ).reshape(epad // SUB, SUB)
    jj = jnp.concatenate(
        [mol_nbrs[:, 1], jnp.full((epad - e,), n, jnp.int32)]
    ).reshape(epad // SUB, SUB)

    mesh = plsc.VectorSubcoreMesh(core_axis_name="c", subcore_axis_name="s")
    edge_call = functools.partial(
        pl.kernel,
        out_type=jax.ShapeDtypeStruct((NW, NMOL), jnp.float32),
        mesh=mesh,
        scratch_types=[
            pltpu.VMEM((GSUB, SUB), jnp.int32),
            pltpu.VMEM((GSUB, SUB), jnp.int32),
            pltpu.VMEM((CHUNK, 16), jnp.float32),
            pltpu.VMEM((CHUNK, 16), jnp.float32),
            pltpu.VMEM((NMOL, L), jnp.float32),
            pltpu.VMEM((NMOL,), jnp.float32),
            pltpu.SemaphoreType.DMA,
            pltpu.SemaphoreType.DMA,
        ],
    )(functools.partial(_edge_body, nchunk))
    partials = edge_call(ii, jj, recs)

    combine = pl.pallas_call(
        _combine_body,
        in_specs=[pl.BlockSpec((NW, NMOL), lambda: (0, 0))],
        out_specs=pl.BlockSpec((1, NMOL), lambda: (0, 0)),
        out_shape=jax.ShapeDtypeStruct((1, NMOL), jnp.float32),
    )
    energy = combine(partials).reshape(NMOL, 1)
    return (energy, q_pad[:n])


# speedup vs baseline: 60.5890x; 60.5890x over previous
import functools

import jax
import jax.numpy as jnp
from jax import lax
from jax.experimental import pallas as pl
from jax.experimental.pallas import tpu as pltpu
from jax.experimental.pallas import tpu_sc as plsc

EPS = 1e-15
BOHR2 = 0.529177 * 0.529177
KE_KCAL = 332.0637
R_ON = 1.25
R_OFF = 3.75
INV_W = 1.0 / (R_OFF - R_ON)

FEAT = 128
NMOL = 448

NC, NS, L = 2, 16, 16
NW = NC * NS

R = 1024
CHUNK = 1024
SUB = 128
GSUB = CHUNK // SUB
GROUPS = CHUNK // L


def _rsqrt(s):
    i = lax.bitcast_convert_type(s, jnp.int32)
    i = 0x5F3759DF - lax.shift_right_arithmetic(i, 1)
    y = lax.bitcast_convert_type(i, jnp.float32)
    for _ in range(3):
        y = y * (1.5 - 0.5 * s * y * y)
    return y


def _charge_body(nblk, f_ref, z_ref, mol_ref, tc_ref, na_ref, w_ref,
                 zt_ref, charge_ref, molsum_ref, corr_ref):
    pid = pl.program_id(0)
    f = f_ref[...]
    zcol = z_ref[...].reshape(R, 1)
    onehot_z = (zcol == lax.broadcasted_iota(jnp.int32, (R, FEAT), 1)
                ).astype(jnp.float32)
    charge = (jnp.sum(f * w_ref[...], axis=1, keepdims=True)
              + jax.lax.dot(onehot_z, zt_ref[...],
                            precision=jax.lax.Precision.HIGHEST))
    charge_ref[...] = charge

    molcol = mol_ref[...].reshape(R, 1)
    onehot_m = (molcol == lax.broadcasted_iota(jnp.int32, (R, NMOL), 1)
                ).astype(jnp.float32)
    part = jax.lax.dot(charge.reshape(1, R), onehot_m,
                       precision=jax.lax.Precision.HIGHEST)

    @pl.when(pid == 0)
    def _():
        molsum_ref[...] = jnp.zeros_like(molsum_ref)

    molsum_ref[...] += part

    @pl.when(pid == nblk - 1)
    def _():
        denom = jnp.maximum(na_ref[...], 1).astype(jnp.float32)
        corr_ref[...] = (tc_ref[...] - molsum_ref[...]) / denom


def _q_body(n_rows, charge_ref, mol_ref, xyz_ref, corr_ref, q_ref, rec_ref):
    pid = pl.program_id(0)
    molcol = mol_ref[...].reshape(R, 1)
    onehot_m = (molcol == lax.broadcasted_iota(jnp.int32, (R, NMOL), 1)
                ).astype(jnp.float32)
    q = charge_ref[...] + jax.lax.dot(
        onehot_m, corr_ref[...].reshape(NMOL, 1),
        precision=jax.lax.Precision.HIGHEST)
    rowid = pid * R + lax.broadcasted_iota(jnp.int32, (R, 1), 0)
    qm = jnp.where(rowid < n_rows, q, 0.0)
    q_ref[...] = qm
    rec_ref[...] = jnp.concatenate(
        [xyz_ref[...], qm, molcol.astype(jnp.float32),
         jnp.zeros((R, 11), jnp.float32)], axis=1)


def _edge_body(nchunk, ii_hbm, jj_hbm, recs_hbm, out_hbm,
               idx_i, idx_j, ri, rj, acc, sem_a, sem_b):
    wid = lax.axis_index("s") * NC + lax.axis_index("c")
    base_row = wid * (nchunk * GSUB)
    lane = lax.iota(jnp.int32, L)

    zeros16 = jnp.zeros((L,), jnp.float32)

    def zero_body(m, carry):
        acc[m] = zeros16
        return carry

    lax.fori_loop(0, NMOL, zero_body, 0)

    def chunk_body(c, carry):
        r0 = base_row + c * GSUB
        pltpu.sync_copy(ii_hbm.at[pl.ds(r0, GSUB)], idx_i)
        pltpu.sync_copy(jj_hbm.at[pl.ds(r0, GSUB)], idx_j)
        cps = []
        for k in range(GSUB):
            cps.append(pltpu.async_copy(
                recs_hbm.at[idx_i.at[k]], ri.at[pl.ds(k * SUB, SUB)], sem_a))
            cps.append(pltpu.async_copy(
                recs_hbm.at[idx_j.at[k]], rj.at[pl.ds(k * SUB, SUB)], sem_b))
        for cp in cps:
            cp.wait()

        def group_body(g, gcarry):
            row = g * L + lane

            def col(ref, cix):
                return plsc.load_gather(ref, [row, lane * 0 + cix])

            xi = col(ri, 0); yi = col(ri, 1); zi = col(ri, 2)
            qi = col(ri, 3); mi = col(ri, 4)
            xj = col(rj, 0); yj = col(rj, 1); zj = col(rj, 2)
            qj = col(rj, 3)

            dx = xi - xj
            dy = yi - yj
            dz = zi - zj
            s = dx * dx + dy * dy + dz * dz + EPS
            t = _rsqrt(s)
            r = s * t
            u = _rsqrt(s + BOHR2)

            x = (r - R_ON) * INV_W
            y = 1.0 - x
            mask = (x > 0.0) & (y > 0.0)
            denom = jnp.where(mask, x * y, 1.0)
            earg = (x - y) / denom
            small = mask & (earg < 34.0)
            safe = jnp.where(small, earg, 0.0)
            mid = jnp.where(earg >= 34.0, 0.0, 1.0 / (1.0 + jnp.exp(safe)))
            fs = jnp.where(mask, mid,
                           jnp.where((x <= 0.0) & (y > 0.0), 1.0, 0.0))

            pw = KE_KCAL * (qi * qj) * (fs * u + (1.0 - fs) * t)
            seg = mi.astype(jnp.int32)
            plsc.addupdate_scatter(acc, [seg, lane], pw)
            return gcarry

        lax.fori_loop(0, GROUPS, group_body, 0)
        return carry

    lax.fori_loop(0, nchunk, chunk_body, 0)
    pltpu.sync_copy(acc, out_hbm.at[wid])


def _combine_body(p_ref, out_ref):
    out_ref[...] = jnp.sum(p_ref[...], axis=(0, 2)).reshape(1, NMOL)


def kernel(f, z, xyz, total_charge, num_atoms, mol_nbrs, W, z_table):
    n = f.shape[0]
    e = mol_nbrs.shape[0]
    npad = ((n + 1 + R - 1) // R) * R
    nblk = npad // R
    per_tile = (e + NW * CHUNK - 1) // (NW * CHUNK) * CHUNK
    epad = per_tile * NW
    nchunk = per_tile // CHUNK

    mol_idx = jnp.repeat(jnp.arange(NMOL, dtype=jnp.int32), num_atoms,
                         total_repeat_length=n)
    f_p = jnp.pad(f, ((0, npad - n), (0, 0)))
    z_p = jnp.pad(z, (0, npad - n)).reshape(nblk, 1, R)
    mol_p = jnp.pad(mol_idx, (0, npad - n)).reshape(nblk, 1, R)
    xyz_p = jnp.pad(xyz, ((0, npad - n), (0, 0)))
    zt_p = jnp.zeros((FEAT, 1), jnp.float32).at[:z_table.shape[0]].set(z_table)
    tc2 = total_charge.reshape(1, NMOL)
    na2 = num_atoms.reshape(1, NMOL)

    grid_a = pl.pallas_call(
        functools.partial(_charge_body, nblk),
        grid=(nblk,),
        in_specs=[
            pl.BlockSpec((R, FEAT), lambda i: (i, 0)),
            pl.BlockSpec((1, 1, R), lambda i: (i, 0, 0)),
            pl.BlockSpec((1, 1, R), lambda i: (i, 0, 0)),
            pl.BlockSpec((1, NMOL), lambda i: (0, 0)),
            pl.BlockSpec((1, NMOL), lambda i: (0, 0)),
            pl.BlockSpec((1, FEAT), lambda i: (0, 0)),
            pl.BlockSpec((FEAT, 1), lambda i: (0, 0)),
        ],
        out_specs=[
            pl.BlockSpec((R, 1), lambda i: (i, 0)),
            pl.BlockSpec((1, NMOL), lambda i: (0, 0)),
            pl.BlockSpec((1, NMOL), lambda i: (0, 0)),
        ],
        out_shape=[
            jax.ShapeDtypeStruct((npad, 1), jnp.float32),
            jax.ShapeDtypeStruct((1, NMOL), jnp.float32),
            jax.ShapeDtypeStruct((1, NMOL), jnp.float32),
        ],
    )
    charge, _, corr = grid_a(f_p, z_p, mol_p, tc2, na2, W, zt_p)

    grid_b = pl.pallas_call(
        functools.partial(_q_body, n),
        grid=(nblk,),
        in_specs=[
            pl.BlockSpec((R, 1), lambda i: (i, 0)),
            pl.BlockSpec((1, 1, R), lambda i: (i, 0, 0)),
            pl.BlockSpec((R, 3), lambda i: (i, 0)),
            pl.BlockSpec((1, NMOL), lambda i: (0, 0)),
        ],
        out_specs=[
            pl.BlockSpec((R, 1), lambda i: (i, 0)),
            pl.BlockSpec((R, 16), lambda i: (i, 0)),
        ],
        out_shape=[
            jax.ShapeDtypeStruct((npad, 1), jnp.float32),
            jax.ShapeDtypeStruct((npad, 16), jnp.float32),
        ],
    )
    q_pad, recs = grid_b(charge, mol_p, xyz_p, corr)

    ii = jnp.concatenate(
        [mol_nbrs[:, 0], jnp.full((epad - e,), n, jnp.int32)]
    ).reshape(epad // SUB, SUB)
    jj = jnp.concatenate(
        [mol_nbrs[:, 1], jnp.full((epad - e,), n, jnp.int32)]
    ).reshape(epad // SUB, SUB)

    mesh = plsc.VectorSubcoreMesh(core_axis_name="c", subcore_axis_name="s")
    edge_call = functools.partial(
        pl.kernel,
        out_type=jax.ShapeDtypeStruct((NW, NMOL, L), jnp.float32),
        mesh=mesh,
        scratch_types=[
            pltpu.VMEM((GSUB, SUB), jnp.int32),
            pltpu.VMEM((GSUB, SUB), jnp.int32),
            pltpu.VMEM((CHUNK, 16), jnp.float32),
            pltpu.VMEM((CHUNK, 16), jnp.float32),
            pltpu.VMEM((NMOL, L), jnp.float32),
            pltpu.SemaphoreType.DMA,
            pltpu.SemaphoreType.DMA,
        ],
        compiler_params=pltpu.CompilerParams(needs_layout_passes=False,
                                             use_tc_tiling_on_sc=False),
    )(functools.partial(_edge_body, nchunk))
    partials = edge_call(ii, jj, recs)

    combine = pl.pallas_call(
        _combine_body,
        in_specs=[pl.BlockSpec((NW, NMOL, L), lambda: (0, 0, 0))],
        out_specs=pl.BlockSpec((1, NMOL), lambda: (0, 0)),
        out_shape=jax.ShapeDtypeStruct((1, NMOL), jnp.float32),
    )
    energy = combine(partials).reshape(NMOL, 1)
    return (energy, q_pad[:n])


# speedup vs baseline: 67.1203x; 1.1078x over previous
import functools

import jax
import jax.numpy as jnp
from jax import lax
from jax.experimental import pallas as pl
from jax.experimental.pallas import tpu as pltpu
from jax.experimental.pallas import tpu_sc as plsc

EPS = 1e-15
BOHR2 = 0.529177 * 0.529177
KE_KCAL = 332.0637
R_ON = 1.25
R_OFF = 3.75
INV_W = 1.0 / (R_OFF - R_ON)

FEAT = 128
NMOL = 448

NC, NS, L = 2, 16, 16
NW = NC * NS

R = 1024
CHUNK = 1024
SUB = 128
GSUB = CHUNK // SUB
GROUPS = CHUNK // L


def _rsqrt(s):
    i = lax.bitcast_convert_type(s, jnp.int32)
    i = 0x5F3759DF - lax.shift_right_arithmetic(i, 1)
    y = lax.bitcast_convert_type(i, jnp.float32)
    for _ in range(3):
        y = y * (1.5 - 0.5 * s * y * y)
    return y


def _charge_body(nblk, f_ref, z_ref, mol_ref, tc_ref, na_ref, w_ref,
                 zt_ref, charge_ref, molsum_ref, corr_ref):
    pid = pl.program_id(0)
    f = f_ref[...]
    zcol = z_ref[...].reshape(R, 1)
    onehot_z = (zcol == lax.broadcasted_iota(jnp.int32, (R, FEAT), 1)
                ).astype(jnp.float32)
    charge = (jnp.sum(f * w_ref[...], axis=1, keepdims=True)
              + jax.lax.dot(onehot_z, zt_ref[...],
                            precision=jax.lax.Precision.HIGHEST))
    charge_ref[...] = charge

    molcol = mol_ref[...].reshape(R, 1)
    onehot_m = (molcol == lax.broadcasted_iota(jnp.int32, (R, NMOL), 1)
                ).astype(jnp.float32)
    part = jax.lax.dot(charge.reshape(1, R), onehot_m,
                       precision=jax.lax.Precision.HIGHEST)

    @pl.when(pid == 0)
    def _():
        molsum_ref[...] = jnp.zeros_like(molsum_ref)

    molsum_ref[...] += part

    @pl.when(pid == nblk - 1)
    def _():
        denom = jnp.maximum(na_ref[...], 1).astype(jnp.float32)
        corr_ref[...] = (tc_ref[...] - molsum_ref[...]) / denom


def _q_body(n_rows, charge_ref, mol_ref, xyz_ref, corr_ref, q_ref, rec_ref):
    pid = pl.program_id(0)
    molcol = mol_ref[...].reshape(R, 1)
    onehot_m = (molcol == lax.broadcasted_iota(jnp.int32, (R, NMOL), 1)
                ).astype(jnp.float32)
    q = charge_ref[...] + jax.lax.dot(
        onehot_m, corr_ref[...].reshape(NMOL, 1),
        precision=jax.lax.Precision.HIGHEST)
    rowid = pid * R + lax.broadcasted_iota(jnp.int32, (R, 1), 0)
    qm = jnp.where(rowid < n_rows, q, 0.0)
    q_ref[...] = qm
    rec_ref[...] = jnp.concatenate(
        [xyz_ref[...], qm, molcol.astype(jnp.float32),
         jnp.zeros((R, 11), jnp.float32)], axis=1)


def _edge_body(nchunk, ii_hbm, jj_hbm, recs_hbm, out_hbm,
               idx_i_a, idx_j_a, idx_i_b, idx_j_b,
               ri_a, rj_a, ri_b, rj_b, acc, sem_a, sem_b):
    wid = lax.axis_index("s") * NC + lax.axis_index("c")
    base_row = wid * (nchunk * GSUB)
    lane = lax.iota(jnp.int32, L)

    zeros16 = jnp.zeros((L,), jnp.float32)

    def zero_body(m, carry):
        acc[m] = zeros16
        return carry

    lax.fori_loop(0, NMOL, zero_body, 0)

    def start(c, idx_i, idx_j, ri, rj, sem):
        r0 = base_row + c * GSUB
        pltpu.sync_copy(ii_hbm.at[pl.ds(r0, GSUB)], idx_i)
        pltpu.sync_copy(jj_hbm.at[pl.ds(r0, GSUB)], idx_j)
        for k in range(GSUB):
            pltpu.async_copy(
                recs_hbm.at[idx_i.at[k]], ri.at[pl.ds(k * SUB, SUB)], sem)
            pltpu.async_copy(
                recs_hbm.at[idx_j.at[k]], rj.at[pl.ds(k * SUB, SUB)], sem)

    def drain(idx_i, idx_j, ri, rj, sem):
        for k in range(GSUB):
            pltpu.make_async_copy(
                recs_hbm.at[idx_i.at[k]], ri.at[pl.ds(k * SUB, SUB)], sem
            ).wait()
            pltpu.make_async_copy(
                recs_hbm.at[idx_j.at[k]], rj.at[pl.ds(k * SUB, SUB)], sem
            ).wait()

    def compute(ri, rj):
        def group_body(g, gcarry):
            row = g * L + lane

            def col(ref, cix):
                return plsc.load_gather(ref, [row, lane * 0 + cix])

            xi = col(ri, 0); yi = col(ri, 1); zi = col(ri, 2)
            qi = col(ri, 3); mi = col(ri, 4)
            xj = col(rj, 0); yj = col(rj, 1); zj = col(rj, 2)
            qj = col(rj, 3)

            dx = xi - xj
            dy = yi - yj
            dz = zi - zj
            s = dx * dx + dy * dy + dz * dz + EPS
            t = _rsqrt(s)
            r = s * t
            u = _rsqrt(s + BOHR2)

            x = (r - R_ON) * INV_W
            y = 1.0 - x
            mask = (x > 0.0) & (y > 0.0)
            denom = jnp.where(mask, x * y, 1.0)
            earg = (x - y) / denom
            small = mask & (earg < 34.0)
            safe = jnp.where(small, earg, 0.0)
            mid = jnp.where(earg >= 34.0, 0.0, 1.0 / (1.0 + jnp.exp(safe)))
            fs = jnp.where(mask, mid,
                           jnp.where((x <= 0.0) & (y > 0.0), 1.0, 0.0))

            pw = KE_KCAL * (qi * qj) * (fs * u + (1.0 - fs) * t)
            seg = mi.astype(jnp.int32)
            plsc.addupdate_scatter(acc, [seg, lane], pw)
            return gcarry

        lax.fori_loop(0, GROUPS, group_body, 0)

    start(0, idx_i_a, idx_j_a, ri_a, rj_a, sem_a)

    def pair_body(c2, carry):
        e = 2 * c2
        start(e + 1, idx_i_b, idx_j_b, ri_b, rj_b, sem_b)
        drain(idx_i_a, idx_j_a, ri_a, rj_a, sem_a)
        compute(ri_a, rj_a)

        @pl.when(e + 2 < nchunk)
        def _():
            start(e + 2, idx_i_a, idx_j_a, ri_a, rj_a, sem_a)

        drain(idx_i_b, idx_j_b, ri_b, rj_b, sem_b)
        compute(ri_b, rj_b)
        return carry

    lax.fori_loop(0, nchunk // 2, pair_body, 0)
    pltpu.sync_copy(acc, out_hbm.at[wid])


def _combine_body(p_ref, out_ref):
    out_ref[...] = jnp.sum(p_ref[...], axis=(0, 2)).reshape(1, NMOL)


def kernel(f, z, xyz, total_charge, num_atoms, mol_nbrs, W, z_table):
    n = f.shape[0]
    e = mol_nbrs.shape[0]
    npad = ((n + 1 + R - 1) // R) * R
    nblk = npad // R
    per_tile = (e + NW * 2 * CHUNK - 1) // (NW * 2 * CHUNK) * 2 * CHUNK
    epad = per_tile * NW
    nchunk = per_tile // CHUNK

    mol_idx = jnp.repeat(jnp.arange(NMOL, dtype=jnp.int32), num_atoms,
                         total_repeat_length=n)
    f_p = jnp.pad(f, ((0, npad - n), (0, 0)))
    z_p = jnp.pad(z, (0, npad - n)).reshape(nblk, 1, R)
    mol_p = jnp.pad(mol_idx, (0, npad - n)).reshape(nblk, 1, R)
    xyz_p = jnp.pad(xyz, ((0, npad - n), (0, 0)))
    zt_p = jnp.zeros((FEAT, 1), jnp.float32).at[:z_table.shape[0]].set(z_table)
    tc2 = total_charge.reshape(1, NMOL)
    na2 = num_atoms.reshape(1, NMOL)

    grid_a = pl.pallas_call(
        functools.partial(_charge_body, nblk),
        grid=(nblk,),
        in_specs=[
            pl.BlockSpec((R, FEAT), lambda i: (i, 0)),
            pl.BlockSpec((1, 1, R), lambda i: (i, 0, 0)),
            pl.BlockSpec((1, 1, R), lambda i: (i, 0, 0)),
            pl.BlockSpec((1, NMOL), lambda i: (0, 0)),
            pl.BlockSpec((1, NMOL), lambda i: (0, 0)),
            pl.BlockSpec((1, FEAT), lambda i: (0, 0)),
            pl.BlockSpec((FEAT, 1), lambda i: (0, 0)),
        ],
        out_specs=[
            pl.BlockSpec((R, 1), lambda i: (i, 0)),
            pl.BlockSpec((1, NMOL), lambda i: (0, 0)),
            pl.BlockSpec((1, NMOL), lambda i: (0, 0)),
        ],
        out_shape=[
            jax.ShapeDtypeStruct((npad, 1), jnp.float32),
            jax.ShapeDtypeStruct((1, NMOL), jnp.float32),
            jax.ShapeDtypeStruct((1, NMOL), jnp.float32),
        ],
    )
    charge, _, corr = grid_a(f_p, z_p, mol_p, tc2, na2, W, zt_p)

    grid_b = pl.pallas_call(
        functools.partial(_q_body, n),
        grid=(nblk,),
        in_specs=[
            pl.BlockSpec((R, 1), lambda i: (i, 0)),
            pl.BlockSpec((1, 1, R), lambda i: (i, 0, 0)),
            pl.BlockSpec((R, 3), lambda i: (i, 0)),
            pl.BlockSpec((1, NMOL), lambda i: (0, 0)),
        ],
        out_specs=[
            pl.BlockSpec((R, 1), lambda i: (i, 0)),
            pl.BlockSpec((R, 16), lambda i: (i, 0)),
        ],
        out_shape=[
            jax.ShapeDtypeStruct((npad, 1), jnp.float32),
            jax.ShapeDtypeStruct((npad, 16), jnp.float32),
        ],
    )
    q_pad, recs = grid_b(charge, mol_p, xyz_p, corr)

    ii = jnp.concatenate(
        [mol_nbrs[:, 0], jnp.full((epad - e,), n, jnp.int32)]
    ).reshape(epad // SUB, SUB)
    jj = jnp.concatenate(
        [mol_nbrs[:, 1], jnp.full((epad - e,), n, jnp.int32)]
    ).reshape(epad // SUB, SUB)

    mesh = plsc.VectorSubcoreMesh(core_axis_name="c", subcore_axis_name="s")
    edge_call = functools.partial(
        pl.kernel,
        out_type=jax.ShapeDtypeStruct((NW, NMOL, L), jnp.float32),
        mesh=mesh,
        scratch_types=[
            pltpu.VMEM((GSUB, SUB), jnp.int32),
            pltpu.VMEM((GSUB, SUB), jnp.int32),
            pltpu.VMEM((GSUB, SUB), jnp.int32),
            pltpu.VMEM((GSUB, SUB), jnp.int32),
            pltpu.VMEM((CHUNK, 16), jnp.float32),
            pltpu.VMEM((CHUNK, 16), jnp.float32),
            pltpu.VMEM((CHUNK, 16), jnp.float32),
            pltpu.VMEM((CHUNK, 16), jnp.float32),
            pltpu.VMEM((NMOL, L), jnp.float32),
            pltpu.SemaphoreType.DMA,
            pltpu.SemaphoreType.DMA,
        ],
        compiler_params=pltpu.CompilerParams(needs_layout_passes=False,
                                             use_tc_tiling_on_sc=False),
    )(functools.partial(_edge_body, nchunk))
    partials = edge_call(ii, jj, recs)

    combine = pl.pallas_call(
        _combine_body,
        in_specs=[pl.BlockSpec((NW, NMOL, L), lambda: (0, 0, 0))],
        out_specs=pl.BlockSpec((1, NMOL), lambda: (0, 0)),
        out_shape=jax.ShapeDtypeStruct((1, NMOL), jnp.float32),
    )
    energy = combine(partials).reshape(NMOL, 1)
    return (energy, q_pad[:n])
